# baseline (speedup 1.0000x reference)
import functools

import jax
import jax.numpy as jnp
from jax import lax
from jax.experimental import pallas as pl
from jax.experimental.pallas import tpu as pltpu
from jax.experimental.pallas import tpu_sc as plsc

VOCAB = 1000000
DIM = 32
BATCH = 16384
SEQ = 20
N_CTX = 16
OUT_SEQ = 1 + N_CTX + (SEQ - 1)

NC = 2
NS = 16
NW = NC * NS
LANES_W = BATCH // NW
CB = 256
NLC = LANES_W // CB
Q_ROWS = VOCAB // 4

QS = 250112
QB = 256
NQB = QS // QB


def _pack_body(t0_ref, t1_ref, t2_ref, t3_ref, out_ref):
    out_ref[...] = jnp.concatenate(
        [jnp.swapaxes(r[...], 0, 1) for r in (t0_ref, t1_ref, t2_ref, t3_ref)],
        axis=1,
    )


_pack_table = pl.pallas_call(
    _pack_body,
    out_shape=jax.ShapeDtypeStruct((QS, 128), jnp.float32),
    grid=(NQB,),
    in_specs=[
        pl.BlockSpec(
            (DIM, QB),
            lambda j, a=a: (0, jnp.minimum(a * NQB + j, VOCAB // QB)),
        )
        for a in range(4)
    ],
    out_specs=pl.BlockSpec((QB, 128), lambda j: (j, 0)),
)

_mesh = plsc.VectorSubcoreMesh(
    core_axis_name="c", subcore_axis_name="s", num_cores=NC, num_subcores=NS
)


@functools.partial(
    pl.kernel,
    out_type=jax.ShapeDtypeStruct((OUT_SEQ, DIM, BATCH), jnp.float32),
    mesh=_mesh,
    compiler_params=pltpu.CompilerParams(needs_layout_passes=False),
    scratch_types=[
        pltpu.VMEM((SEQ, CB), jnp.int32),
        pltpu.VMEM((SEQ * CB,), jnp.int32),
        pltpu.VMEM((SEQ, CB), jnp.int32),
        pltpu.VMEM((CB, 128), jnp.float32),
        pltpu.VMEM((CB, 128), jnp.float32),
        pltpu.VMEM((DIM, CB), jnp.float32),
        pltpu.VMEM((DIM, CB), jnp.float32),
        pltpu.VMEM((DIM, CB), jnp.float32),
        pltpu.VMEM((N_CTX, DIM), jnp.float32),
        pltpu.SemaphoreType.DMA,
        pltpu.SemaphoreType.DMA,
        pltpu.SemaphoreType.DMA,
        pltpu.SemaphoreType.DMA,
    ],
)
def _sc_prompt_kernel(
    idx_hbm,
    t4_hbm,
    ctx_hbm,
    out_hbm,
    idx_v,
    q_v,
    subcol_v,
    gbuf_a,
    gbuf_b,
    pbuf_a,
    pbuf_b,
    cbuf,
    ctx_v,
    sem_g0,
    sem_g1,
    sem_w0,
    sem_w1,
):
    wid = lax.axis_index("s") * NC + lax.axis_index("c")
    b0w = wid * LANES_W
    iota16 = lax.iota(jnp.int32, 16)

    gbuf = (gbuf_a, gbuf_b)
    pbuf = (pbuf_a, pbuf_b)
    sem_g = (sem_g0, sem_g1)
    sem_w = (sem_w0, sem_w1)

    pltpu.sync_copy(ctx_hbm, ctx_v)

    def ctx_planes():
        def ctx_plane(j, carry):
            def fill(kb, carry2):
                k0 = kb * 16
                jvec = jnp.full((16,), 0, jnp.int32) + j
                for d in range(DIM):
                    v = plsc.load_gather(
                        ctx_v, [jvec, jnp.full((16,), d, jnp.int32)]
                    )
                    cbuf[d, pl.ds(k0, 16)] = v
                return carry2

            lax.fori_loop(0, CB // 16, fill, 0)
            cps = [
                pltpu.make_async_copy(
                    cbuf,
                    out_hbm.at[1 + j, :, pl.ds(b0w + mm * CB, CB)],
                    sem_w0,
                )
                for mm in range(NLC)
            ]
            for cp in cps:
                cp.start()
            for cp in cps:
                cp.wait()
            return carry

        lax.fori_loop(0, N_CTX, ctx_plane, 0)

    def gather_cp(s, slot):
        qoff = pl.multiple_of(s * CB, 128)
        return pltpu.make_async_copy(
            t4_hbm.at[q_v.at[pl.ds(qoff, CB)]], gbuf[slot], sem_g[slot]
        )

    def write_cp(s, slot, b0):
        pos = jnp.where(s == 0, 0, N_CTX + s)
        return pltpu.make_async_copy(
            pbuf[slot], out_hbm.at[pos, :, pl.ds(b0, CB)], sem_w[slot]
        )

    def extract_to(s, slot):
        def extract(kb, carry2):
            k0 = kb * 16
            rows = iota16 + k0
            subc = subcol_v[s, pl.ds(k0, 16)]
            for d in range(DIM):
                val = plsc.load_gather(gbuf[slot], [rows, subc + d])
                pbuf[slot][d, pl.ds(k0, 16)] = val
            return carry2

        lax.fori_loop(0, CB // 16, extract, 0)

    for m in range(NLC):
        b0 = b0w + m * CB
        pltpu.sync_copy(idx_hbm.at[:, pl.ds(b0, CB)], idx_v)

        def qcalc(t, carry):
            r = t // (CB // 16)
            c0 = (t % (CB // 16)) * 16
            v = idx_v[r, pl.ds(c0, 16)]
            a = v // QS
            q_v[pl.ds(t * 16, 16)] = v - a * QS
            subcol_v[r, pl.ds(c0, 16)] = jnp.left_shift(a, 5)
            return carry

        lax.fori_loop(0, SEQ * (CB // 16), qcalc, 0)

        gather_cp(0, 0).start()
        if m == 0:
            ctx_planes()

        def pair(s2, carry):
            s0 = 2 * s2
            s1 = s0 + 1
            gather_cp(s0, 0).wait()
            gather_cp(s1, 1).start()

            @pl.when(s2 > 0)
            def _():
                write_cp(s0 - 2, 0, b0).wait()

            extract_to(s0, 0)
            write_cp(s0, 0, b0).start()

            gather_cp(s1, 1).wait()

            @pl.when(s2 < SEQ // 2 - 1)
            def _():
                gather_cp(s0 + 2, 0).start()

            @pl.when(s2 > 0)
            def _():
                write_cp(s1 - 2, 1, b0).wait()

            extract_to(s1, 1)
            write_cp(s1, 1, b0).start()
            return carry

        lax.fori_loop(0, SEQ // 2, pair, 0)
        write_cp(SEQ - 2, 0, b0).wait()
        write_cp(SEQ - 1, 1, b0).wait()


def kernel(indices, table, ctx):
    idx_t = indices.T
    tt = table.T
    t4 = _pack_table(tt, tt, tt, tt)
    out_t = _sc_prompt_kernel(idx_t, t4, ctx)
    return out_t.transpose(2, 0, 1)


# speedup vs baseline: 1.2392x; 1.2392x over previous
import functools

import jax
import jax.numpy as jnp
from jax import lax
from jax.experimental import pallas as pl
from jax.experimental.pallas import tpu as pltpu
from jax.experimental.pallas import tpu_sc as plsc

VOCAB = 1000000
DIM = 32
BATCH = 16384
SEQ = 20
N_CTX = 16
OUT_SEQ = 1 + N_CTX + (SEQ - 1)

NC = 2
NS = 16
NW = NC * NS
LANES_W = BATCH // NW
CB = 256
NLC = LANES_W // CB
Q_ROWS = VOCAB // 4

_mesh = plsc.VectorSubcoreMesh(
    core_axis_name="c", subcore_axis_name="s", num_cores=NC, num_subcores=NS
)


@functools.partial(
    pl.kernel,
    out_type=jax.ShapeDtypeStruct((OUT_SEQ, DIM, BATCH), jnp.float32),
    mesh=_mesh,
    compiler_params=pltpu.CompilerParams(needs_layout_passes=False),
    scratch_types=[
        pltpu.VMEM((SEQ, CB), jnp.int32),
        pltpu.VMEM((SEQ * CB,), jnp.int32),
        pltpu.VMEM((SEQ, CB), jnp.int32),
        pltpu.VMEM((CB, 128), jnp.float32),
        pltpu.VMEM((CB, 128), jnp.float32),
        pltpu.VMEM((DIM, CB), jnp.float32),
        pltpu.VMEM((DIM, CB), jnp.float32),
        pltpu.VMEM((DIM, CB), jnp.float32),
        pltpu.VMEM((N_CTX, DIM), jnp.float32),
        pltpu.SemaphoreType.DMA,
        pltpu.SemaphoreType.DMA,
        pltpu.SemaphoreType.DMA,
        pltpu.SemaphoreType.DMA,
    ],
)
def _sc_prompt_kernel(
    idx_hbm,
    t4_hbm,
    ctx_hbm,
    out_hbm,
    idx_v,
    q_v,
    subcol_v,
    gbuf_a,
    gbuf_b,
    pbuf_a,
    pbuf_b,
    cbuf,
    ctx_v,
    sem_g0,
    sem_g1,
    sem_w0,
    sem_w1,
):
    wid = lax.axis_index("s") * NC + lax.axis_index("c")
    b0w = wid * LANES_W
    iota16 = lax.iota(jnp.int32, 16)

    gbuf = (gbuf_a, gbuf_b)
    pbuf = (pbuf_a, pbuf_b)
    sem_g = (sem_g0, sem_g1)
    sem_w = (sem_w0, sem_w1)

    pltpu.sync_copy(ctx_hbm, ctx_v)

    def ctx_planes():
        def ctx_plane(j, carry):
            def fill(kb, carry2):
                k0 = kb * 16
                jvec = jnp.full((16,), 0, jnp.int32) + j
                for d in range(DIM):
                    v = plsc.load_gather(
                        ctx_v, [jvec, jnp.full((16,), d, jnp.int32)]
                    )
                    cbuf[d, pl.ds(k0, 16)] = v
                return carry2

            lax.fori_loop(0, CB // 16, fill, 0)
            cps = [
                pltpu.make_async_copy(
                    cbuf,
                    out_hbm.at[1 + j, :, pl.ds(b0w + mm * CB, CB)],
                    sem_w0,
                )
                for mm in range(NLC)
            ]
            for cp in cps:
                cp.start()
            for cp in cps:
                cp.wait()
            return carry

        lax.fori_loop(0, N_CTX, ctx_plane, 0)

    def gather_cp(s, slot):
        qoff = pl.multiple_of(s * CB, 128)
        return pltpu.make_async_copy(
            t4_hbm.at[q_v.at[pl.ds(qoff, CB)]], gbuf[slot], sem_g[slot]
        )

    def write_cp(s, slot, b0):
        pos = jnp.where(s == 0, 0, N_CTX + s)
        return pltpu.make_async_copy(
            pbuf[slot], out_hbm.at[pos, :, pl.ds(b0, CB)], sem_w[slot]
        )

    def extract_to(s, slot):
        def extract(kb, carry2):
            k0 = kb * 16
            rows = iota16 + k0
            subc = subcol_v[s, pl.ds(k0, 16)]
            for d in range(DIM):
                val = plsc.load_gather(gbuf[slot], [rows, subc + d])
                pbuf[slot][d, pl.ds(k0, 16)] = val
            return carry2

        lax.fori_loop(0, CB // 16, extract, 0)

    for m in range(NLC):
        b0 = b0w + m * CB
        pltpu.sync_copy(idx_hbm.at[:, pl.ds(b0, CB)], idx_v)

        def qcalc(t, carry):
            r = t // (CB // 16)
            c0 = (t % (CB // 16)) * 16
            v = idx_v[r, pl.ds(c0, 16)]
            q_v[pl.ds(t * 16, 16)] = jnp.right_shift(v, 2)
            subcol_v[r, pl.ds(c0, 16)] = jnp.left_shift(
                jnp.bitwise_and(v, 3), 5
            )
            return carry

        lax.fori_loop(0, SEQ * (CB // 16), qcalc, 0)

        gather_cp(0, 0).start()
        if m == 0:
            ctx_planes()

        def pair(s2, carry):
            s0 = 2 * s2
            s1 = s0 + 1
            gather_cp(s0, 0).wait()
            gather_cp(s1, 1).start()

            @pl.when(s2 > 0)
            def _():
                write_cp(s0 - 2, 0, b0).wait()

            extract_to(s0, 0)
            write_cp(s0, 0, b0).start()

            gather_cp(s1, 1).wait()

            @pl.when(s2 < SEQ // 2 - 1)
            def _():
                gather_cp(s0 + 2, 0).start()

            @pl.when(s2 > 0)
            def _():
                write_cp(s1 - 2, 1, b0).wait()

            extract_to(s1, 1)
            write_cp(s1, 1, b0).start()
            return carry

        lax.fori_loop(0, SEQ // 2, pair, 0)
        write_cp(SEQ - 2, 0, b0).wait()
        write_cp(SEQ - 1, 1, b0).wait()


def kernel(indices, table, ctx):
    idx_t = indices.T
    t4 = table.reshape(Q_ROWS, 128)
    out_t = _sc_prompt_kernel(idx_t, t4, ctx)
    return out_t.transpose(2, 0, 1)


# speedup vs baseline: 1.6673x; 1.3455x over previous
import functools

import jax
import jax.numpy as jnp
from jax import lax
from jax.experimental import pallas as pl
from jax.experimental.pallas import tpu as pltpu
from jax.experimental.pallas import tpu_sc as plsc

VOCAB = 1000000
DIM = 32
BATCH = 16384
SEQ = 20
N_CTX = 16
OUT_SEQ = 1 + N_CTX + (SEQ - 1)

NC = 2
NS = 16
NW = NC * NS
LANES_W = BATCH // NW
CB = 256
NLC = LANES_W // CB
Q_ROWS = VOCAB // 4

QS = 251904
QB = 2048
NQB = QS // QB


def _pack_body(t0_ref, t1_ref, t2_ref, t3_ref, out_ref):
    out_ref[...] = jnp.concatenate(
        [jnp.swapaxes(r[...], 0, 1) for r in (t0_ref, t1_ref, t2_ref, t3_ref)],
        axis=1,
    )


_pack_table = pl.pallas_call(
    _pack_body,
    out_shape=jax.ShapeDtypeStruct((QS, 128), jnp.float32),
    grid=(NQB,),
    in_specs=[
        pl.BlockSpec(
            (DIM, QB),
            lambda j, a=a: (0, jnp.minimum(a * NQB + j, VOCAB // QB)),
        )
        for a in range(4)
    ],
    out_specs=pl.BlockSpec((QB, 128), lambda j: (j, 0)),
)


_mesh = plsc.VectorSubcoreMesh(
    core_axis_name="c", subcore_axis_name="s", num_cores=NC, num_subcores=NS
)


@functools.partial(
    pl.kernel,
    out_type=jax.ShapeDtypeStruct((OUT_SEQ, DIM, BATCH), jnp.float32),
    mesh=_mesh,
    compiler_params=pltpu.CompilerParams(needs_layout_passes=False),
    scratch_types=[
        pltpu.VMEM((SEQ, CB), jnp.int32),
        pltpu.VMEM((SEQ * CB,), jnp.int32),
        pltpu.VMEM((SEQ, CB), jnp.int32),
        pltpu.VMEM((CB, 128), jnp.float32),
        pltpu.VMEM((CB, 128), jnp.float32),
        pltpu.VMEM((DIM, CB), jnp.float32),
        pltpu.VMEM((DIM, CB), jnp.float32),
        pltpu.VMEM((DIM, CB), jnp.float32),
        pltpu.VMEM((N_CTX, DIM), jnp.float32),
        pltpu.SemaphoreType.DMA,
        pltpu.SemaphoreType.DMA,
        pltpu.SemaphoreType.DMA,
        pltpu.SemaphoreType.DMA,
    ],
)
def _sc_prompt_kernel(
    idx_hbm,
    t4_hbm,
    ctx_hbm,
    out_hbm,
    idx_v,
    q_v,
    subcol_v,
    gbuf_a,
    gbuf_b,
    pbuf_a,
    pbuf_b,
    cbuf,
    ctx_v,
    sem_g0,
    sem_g1,
    sem_w0,
    sem_w1,
):
    wid = lax.axis_index("s") * NC + lax.axis_index("c")
    b0w = wid * LANES_W
    iota16 = lax.iota(jnp.int32, 16)

    gbuf = (gbuf_a, gbuf_b)
    pbuf = (pbuf_a, pbuf_b)
    sem_g = (sem_g0, sem_g1)
    sem_w = (sem_w0, sem_w1)

    pltpu.sync_copy(ctx_hbm, ctx_v)

    def ctx_planes():
        def ctx_plane(j, carry):
            def fill(kb, carry2):
                k0 = kb * 16
                jvec = jnp.full((16,), 0, jnp.int32) + j
                for d in range(DIM):
                    v = plsc.load_gather(
                        ctx_v, [jvec, jnp.full((16,), d, jnp.int32)]
                    )
                    cbuf[d, pl.ds(k0, 16)] = v
                return carry2

            lax.fori_loop(0, CB // 16, fill, 0)
            cps = [
                pltpu.make_async_copy(
                    cbuf,
                    out_hbm.at[1 + j, :, pl.ds(b0w + mm * CB, CB)],
                    sem_w0,
                )
                for mm in range(NLC)
            ]
            for cp in cps:
                cp.start()
            for cp in cps:
                cp.wait()
            return carry

        lax.fori_loop(0, N_CTX, ctx_plane, 0)

    def gather_cp(s, slot):
        qoff = pl.multiple_of(s * CB, 128)
        return pltpu.make_async_copy(
            t4_hbm.at[q_v.at[pl.ds(qoff, CB)]], gbuf[slot], sem_g[slot]
        )

    def write_cp(s, slot, b0):
        pos = jnp.where(s == 0, 0, N_CTX + s)
        return pltpu.make_async_copy(
            pbuf[slot], out_hbm.at[pos, :, pl.ds(b0, CB)], sem_w[slot]
        )

    def extract_to(s, slot):
        def extract(kb, carry2):
            k0 = kb * 16
            rows = iota16 + k0
            subc = subcol_v[s, pl.ds(k0, 16)]
            for d in range(DIM):
                val = plsc.load_gather(gbuf[slot], [rows, subc + d])
                pbuf[slot][d, pl.ds(k0, 16)] = val
            return carry2

        lax.fori_loop(0, CB // 16, extract, 0)

    for m in range(NLC):
        b0 = b0w + m * CB
        pltpu.sync_copy(idx_hbm.at[:, pl.ds(b0, CB)], idx_v)

        def qcalc(t, carry):
            r = t // (CB // 16)
            c0 = (t % (CB // 16)) * 16
            v = idx_v[r, pl.ds(c0, 16)]
            a = v // QS
            q_v[pl.ds(t * 16, 16)] = v - a * QS
            subcol_v[r, pl.ds(c0, 16)] = jnp.left_shift(a, 5)
            return carry

        lax.fori_loop(0, SEQ * (CB // 16), qcalc, 0)

        gather_cp(0, 0).start()
        if m == 0:
            ctx_planes()

        def pair(s2, carry):
            s0 = 2 * s2
            s1 = s0 + 1
            gather_cp(s0, 0).wait()
            gather_cp(s1, 1).start()

            @pl.when(s2 > 0)
            def _():
                write_cp(s0 - 2, 0, b0).wait()

            extract_to(s0, 0)
            write_cp(s0, 0, b0).start()

            gather_cp(s1, 1).wait()

            @pl.when(s2 < SEQ // 2 - 1)
            def _():
                gather_cp(s0 + 2, 0).start()

            @pl.when(s2 > 0)
            def _():
                write_cp(s1 - 2, 1, b0).wait()

            extract_to(s1, 1)
            write_cp(s1, 1, b0).start()
            return carry

        lax.fori_loop(0, SEQ // 2, pair, 0)
        write_cp(SEQ - 2, 0, b0).wait()
        write_cp(SEQ - 1, 1, b0).wait()


def kernel(indices, table, ctx):
    idx_t = indices.T
    tt = table.T
    t4 = _pack_table(tt, tt, tt, tt)
    out_t = _sc_prompt_kernel(idx_t, t4, ctx)
    return out_t.transpose(2, 0, 1)


# speedup vs baseline: 1.6888x; 1.0129x over previous
import functools

import jax
import jax.numpy as jnp
from jax import lax
from jax.experimental import pallas as pl
from jax.experimental.pallas import tpu as pltpu
from jax.experimental.pallas import tpu_sc as plsc

VOCAB = 1000000
DIM = 32
BATCH = 16384
SEQ = 20
N_CTX = 16
OUT_SEQ = 1 + N_CTX + (SEQ - 1)

NC = 2
NS = 16
NW = NC * NS
LANES_W = BATCH // NW
CB = 256
NLC = LANES_W // CB
Q_ROWS = VOCAB // 4

QS = 253952
QB = 4096
NQB = QS // QB


def _pack_body(t0_ref, t1_ref, t2_ref, t3_ref, out_ref):
    out_ref[...] = jnp.concatenate(
        [jnp.swapaxes(r[...], 0, 1) for r in (t0_ref, t1_ref, t2_ref, t3_ref)],
        axis=1,
    )


_pack_table = pl.pallas_call(
    _pack_body,
    out_shape=jax.ShapeDtypeStruct((QS, 128), jnp.float32),
    grid=(NQB,),
    in_specs=[
        pl.BlockSpec(
            (DIM, QB),
            lambda j, a=a: (0, jnp.minimum(a * NQB + j, VOCAB // QB)),
        )
        for a in range(4)
    ],
    out_specs=pl.BlockSpec((QB, 128), lambda j: (j, 0)),
)


_mesh = plsc.VectorSubcoreMesh(
    core_axis_name="c", subcore_axis_name="s", num_cores=NC, num_subcores=NS
)


@functools.partial(
    pl.kernel,
    out_type=jax.ShapeDtypeStruct((OUT_SEQ, DIM, BATCH), jnp.float32),
    mesh=_mesh,
    compiler_params=pltpu.CompilerParams(needs_layout_passes=False),
    scratch_types=[
        pltpu.VMEM((SEQ, CB), jnp.int32),
        pltpu.VMEM((SEQ * CB,), jnp.int32),
        pltpu.VMEM((SEQ, CB), jnp.int32),
        pltpu.VMEM((CB, 128), jnp.float32),
        pltpu.VMEM((CB, 128), jnp.float32),
        pltpu.VMEM((DIM, CB), jnp.float32),
        pltpu.VMEM((DIM, CB), jnp.float32),
        pltpu.VMEM((DIM, CB), jnp.float32),
        pltpu.VMEM((N_CTX, DIM), jnp.float32),
        pltpu.SemaphoreType.DMA,
        pltpu.SemaphoreType.DMA,
        pltpu.SemaphoreType.DMA,
        pltpu.SemaphoreType.DMA,
    ],
)
def _sc_prompt_kernel(
    idx_hbm,
    t4_hbm,
    ctx_hbm,
    out_hbm,
    idx_v,
    q_v,
    subcol_v,
    gbuf_a,
    gbuf_b,
    pbuf_a,
    pbuf_b,
    cbuf,
    ctx_v,
    sem_g0,
    sem_g1,
    sem_w0,
    sem_w1,
):
    wid = lax.axis_index("s") * NC + lax.axis_index("c")
    b0w = wid * LANES_W
    iota16 = lax.iota(jnp.int32, 16)

    gbuf = (gbuf_a, gbuf_b)
    pbuf = (pbuf_a, pbuf_b)
    sem_g = (sem_g0, sem_g1)
    sem_w = (sem_w0, sem_w1)

    pltpu.sync_copy(ctx_hbm, ctx_v)

    def ctx_planes():
        def ctx_plane(j, carry):
            def fill(kb, carry2):
                k0 = kb * 16
                jvec = jnp.full((16,), 0, jnp.int32) + j
                for d in range(DIM):
                    v = plsc.load_gather(
                        ctx_v, [jvec, jnp.full((16,), d, jnp.int32)]
                    )
                    cbuf[d, pl.ds(k0, 16)] = v
                return carry2

            lax.fori_loop(0, CB // 16, fill, 0)
            cps = [
                pltpu.make_async_copy(
                    cbuf,
                    out_hbm.at[1 + j, :, pl.ds(b0w + mm * CB, CB)],
                    sem_w0,
                )
                for mm in range(NLC)
            ]
            for cp in cps:
                cp.start()
            for cp in cps:
                cp.wait()
            return carry

        lax.fori_loop(0, N_CTX, ctx_plane, 0)

    def gather_cp(s, slot):
        qoff = pl.multiple_of(s * CB, 128)
        return pltpu.make_async_copy(
            t4_hbm.at[q_v.at[pl.ds(qoff, CB)]], gbuf[slot], sem_g[slot]
        )

    def write_cp(s, slot, b0):
        pos = jnp.where(s == 0, 0, N_CTX + s)
        return pltpu.make_async_copy(
            pbuf[slot], out_hbm.at[pos, :, pl.ds(b0, CB)], sem_w[slot]
        )

    def extract_to(s, slot):
        def extract(kb, carry2):
            k0 = kb * 16
            rows = iota16 + k0
            subc = subcol_v[s, pl.ds(k0, 16)]
            for d in range(DIM):
                val = plsc.load_gather(gbuf[slot], [rows, subc + d])
                pbuf[slot][d, pl.ds(k0, 16)] = val
            return carry2

        lax.fori_loop(0, CB // 16, extract, 0)

    for m in range(NLC):
        b0 = b0w + m * CB
        pltpu.sync_copy(idx_hbm.at[:, pl.ds(b0, CB)], idx_v)

        def qcalc(t, carry):
            r = t // (CB // 16)
            c0 = (t % (CB // 16)) * 16
            v = idx_v[r, pl.ds(c0, 16)]
            a = v // QS
            q_v[pl.ds(t * 16, 16)] = v - a * QS
            subcol_v[r, pl.ds(c0, 16)] = jnp.left_shift(a, 5)
            return carry

        lax.fori_loop(0, SEQ * (CB // 16), qcalc, 0)

        gather_cp(0, 0).start()
        if m == 0:
            ctx_planes()

        def pair(s2, carry):
            s0 = 2 * s2
            s1 = s0 + 1
            gather_cp(s0, 0).wait()
            gather_cp(s1, 1).start()

            @pl.when(s2 > 0)
            def _():
                write_cp(s0 - 2, 0, b0).wait()

            extract_to(s0, 0)
            write_cp(s0, 0, b0).start()

            gather_cp(s1, 1).wait()

            @pl.when(s2 < SEQ // 2 - 1)
            def _():
                gather_cp(s0 + 2, 0).start()

            @pl.when(s2 > 0)
            def _():
                write_cp(s1 - 2, 1, b0).wait()

            extract_to(s1, 1)
            write_cp(s1, 1, b0).start()
            return carry

        lax.fori_loop(0, SEQ // 2, pair, 0)
        write_cp(SEQ - 2, 0, b0).wait()
        write_cp(SEQ - 1, 1, b0).wait()


def kernel(indices, table, ctx):
    idx_t = indices.T
    tt = table.T
    t4 = _pack_table(tt, tt, tt, tt)
    out_t = _sc_prompt_kernel(idx_t, t4, ctx)
    return out_t.transpose(2, 0, 1)


# speedup vs baseline: 1.6999x; 1.0066x over previous
import functools

import jax
import jax.numpy as jnp
from jax import lax
from jax.experimental import pallas as pl
from jax.experimental.pallas import tpu as pltpu
from jax.experimental.pallas import tpu_sc as plsc

VOCAB = 1000000
DIM = 32
BATCH = 16384
SEQ = 20
N_CTX = 16
OUT_SEQ = 1 + N_CTX + (SEQ - 1)

NC = 2
NS = 16
NW = NC * NS
LANES_W = BATCH // NW
CB = 256
NLC = LANES_W // CB
Q_ROWS = VOCAB // 4

QS = 253952
QB = 8192
NQB = QS // QB


def _pack_body(t0_ref, t1_ref, t2_ref, t3_ref, out_ref):
    out_ref[...] = jnp.concatenate(
        [jnp.swapaxes(r[...], 0, 1) for r in (t0_ref, t1_ref, t2_ref, t3_ref)],
        axis=1,
    )


_pack_table = pl.pallas_call(
    _pack_body,
    out_shape=jax.ShapeDtypeStruct((QS, 128), jnp.float32),
    grid=(NQB,),
    in_specs=[
        pl.BlockSpec(
            (DIM, QB),
            lambda j, a=a: (0, jnp.minimum(a * NQB + j, VOCAB // QB)),
        )
        for a in range(4)
    ],
    out_specs=pl.BlockSpec((QB, 128), lambda j: (j, 0)),
)


_mesh = plsc.VectorSubcoreMesh(
    core_axis_name="c", subcore_axis_name="s", num_cores=NC, num_subcores=NS
)


@functools.partial(
    pl.kernel,
    out_type=jax.ShapeDtypeStruct((OUT_SEQ, DIM, BATCH), jnp.float32),
    mesh=_mesh,
    compiler_params=pltpu.CompilerParams(needs_layout_passes=False),
    scratch_types=[
        pltpu.VMEM((SEQ, CB), jnp.int32),
        pltpu.VMEM((SEQ * CB,), jnp.int32),
        pltpu.VMEM((SEQ, CB), jnp.int32),
        pltpu.VMEM((CB, 128), jnp.float32),
        pltpu.VMEM((CB, 128), jnp.float32),
        pltpu.VMEM((DIM, CB), jnp.float32),
        pltpu.VMEM((DIM, CB), jnp.float32),
        pltpu.VMEM((DIM, CB), jnp.float32),
        pltpu.VMEM((N_CTX, DIM), jnp.float32),
        pltpu.SemaphoreType.DMA,
        pltpu.SemaphoreType.DMA,
        pltpu.SemaphoreType.DMA,
        pltpu.SemaphoreType.DMA,
    ],
)
def _sc_prompt_kernel(
    idx_hbm,
    t4_hbm,
    ctx_hbm,
    out_hbm,
    idx_v,
    q_v,
    subcol_v,
    gbuf_a,
    gbuf_b,
    pbuf_a,
    pbuf_b,
    cbuf,
    ctx_v,
    sem_g0,
    sem_g1,
    sem_w0,
    sem_w1,
):
    wid = lax.axis_index("s") * NC + lax.axis_index("c")
    b0w = wid * LANES_W
    iota16 = lax.iota(jnp.int32, 16)

    gbuf = (gbuf_a, gbuf_b)
    pbuf = (pbuf_a, pbuf_b)
    sem_g = (sem_g0, sem_g1)
    sem_w = (sem_w0, sem_w1)

    pltpu.sync_copy(ctx_hbm, ctx_v)

    def ctx_planes():
        def ctx_plane(j, carry):
            def fill(kb, carry2):
                k0 = kb * 16
                jvec = jnp.full((16,), 0, jnp.int32) + j
                for d in range(DIM):
                    v = plsc.load_gather(
                        ctx_v, [jvec, jnp.full((16,), d, jnp.int32)]
                    )
                    cbuf[d, pl.ds(k0, 16)] = v
                return carry2

            lax.fori_loop(0, CB // 16, fill, 0)
            cps = [
                pltpu.make_async_copy(
                    cbuf,
                    out_hbm.at[1 + j, :, pl.ds(b0w + mm * CB, CB)],
                    sem_w0,
                )
                for mm in range(NLC)
            ]
            for cp in cps:
                cp.start()
            for cp in cps:
                cp.wait()
            return carry

        lax.fori_loop(0, N_CTX, ctx_plane, 0)

    def gather_cp(s, slot):
        qoff = pl.multiple_of(s * CB, 128)
        return pltpu.make_async_copy(
            t4_hbm.at[q_v.at[pl.ds(qoff, CB)]], gbuf[slot], sem_g[slot]
        )

    def write_cp(s, slot, b0):
        pos = jnp.where(s == 0, 0, N_CTX + s)
        return pltpu.make_async_copy(
            pbuf[slot], out_hbm.at[pos, :, pl.ds(b0, CB)], sem_w[slot]
        )

    def extract_to(s, slot):
        def extract(kb, carry2):
            k0 = kb * 16
            rows = iota16 + k0
            subc = subcol_v[s, pl.ds(k0, 16)]
            for d in range(DIM):
                val = plsc.load_gather(gbuf[slot], [rows, subc + d])
                pbuf[slot][d, pl.ds(k0, 16)] = val
            return carry2

        lax.fori_loop(0, CB // 16, extract, 0)

    for m in range(NLC):
        b0 = b0w + m * CB
        pltpu.sync_copy(idx_hbm.at[:, pl.ds(b0, CB)], idx_v)

        def qcalc(t, carry):
            r = t // (CB // 16)
            c0 = (t % (CB // 16)) * 16
            v = idx_v[r, pl.ds(c0, 16)]
            a = v // QS
            q_v[pl.ds(t * 16, 16)] = v - a * QS
            subcol_v[r, pl.ds(c0, 16)] = jnp.left_shift(a, 5)
            return carry

        lax.fori_loop(0, SEQ * (CB // 16), qcalc, 0)

        gather_cp(0, 0).start()
        if m == 0:
            ctx_planes()

        def pair(s2, carry):
            s0 = 2 * s2
            s1 = s0 + 1
            gather_cp(s0, 0).wait()
            gather_cp(s1, 1).start()

            @pl.when(s2 > 0)
            def _():
                write_cp(s0 - 2, 0, b0).wait()

            extract_to(s0, 0)
            write_cp(s0, 0, b0).start()

            gather_cp(s1, 1).wait()

            @pl.when(s2 < SEQ // 2 - 1)
            def _():
                gather_cp(s0 + 2, 0).start()

            @pl.when(s2 > 0)
            def _():
                write_cp(s1 - 2, 1, b0).wait()

            extract_to(s1, 1)
            write_cp(s1, 1, b0).start()
            return carry

        lax.fori_loop(0, SEQ // 2, pair, 0)
        write_cp(SEQ - 2, 0, b0).wait()
        write_cp(SEQ - 1, 1, b0).wait()


def kernel(indices, table, ctx):
    idx_t = indices.T
    tt = table.T
    t4 = _pack_table(tt, tt, tt, tt)
    out_t = _sc_prompt_kernel(idx_t, t4, ctx)
    return out_t.transpose(2, 0, 1)
